# baseline (speedup 1.0000x reference)
import functools

import jax
import jax.numpy as jnp
from jax import lax
from jax.experimental import pallas as pl
from jax.experimental.pallas import tpu as pltpu
from jax.experimental.pallas import tpu_sc as plsc

TOTAL_C = 256

NC = 2
NS = 16
NW = NC * NS

CHUNK = 16


def _sc_scatter(x2, dst_idx, pad_idx, zrows, n_rows, n_pad_rows, hw):
    rows_per_w = n_rows // NW
    prows_per_w = n_pad_rows // NW
    n_chunks = rows_per_w // CHUNK
    n_pchunks = prows_per_w // CHUNK

    mesh = plsc.VectorSubcoreMesh(core_axis_name="c", subcore_axis_name="s")

    @functools.partial(
        pl.kernel,
        mesh=mesh,
        compiler_params=pltpu.CompilerParams(use_tc_tiling_on_sc=False),
        out_type=jax.ShapeDtypeStruct((n_rows + n_pad_rows, hw), jnp.float32),
        scratch_types=[
            pltpu.VMEM((n_chunks, CHUNK), jnp.int32),
            pltpu.VMEM((n_pchunks, CHUNK), jnp.int32),
            pltpu.VMEM((CHUNK, hw), jnp.float32),
            pltpu.SemaphoreType.DMA,
        ],
    )
    def k(x_hbm, dsti_hbm, padi_hbm, z_hbm, out_hbm, idx_v, pidx_v, buf, sem):
        wid = lax.axis_index("s") * NC + lax.axis_index("c")
        pltpu.sync_copy(dsti_hbm.at[wid], idx_v)
        pltpu.sync_copy(padi_hbm.at[wid], pidx_v)
        for j in range(n_chunks):
            pltpu.sync_copy(
                x_hbm.at[pl.ds(wid * rows_per_w + j * CHUNK, CHUNK)], buf)
            pltpu.async_copy(buf, out_hbm.at[idx_v.at[j]], sem).wait()
        pltpu.sync_copy(z_hbm, buf)
        for j in range(n_pchunks):
            pltpu.async_copy(buf, out_hbm.at[pidx_v.at[j]], sem).wait()

    return k(x2, dst_idx, pad_idx, zrows)


def kernel(x, conv_forward_indices):
    b, c_in, h, w = x.shape
    hw = h * w
    idx = conv_forward_indices.astype(jnp.int32)

    base = jnp.arange(b, dtype=jnp.int32)[:, None] * TOTAL_C
    dst_rows = (base + idx[None, :]).reshape(NW, -1, CHUNK)

    covered = jnp.zeros((TOTAL_C,), jnp.bool_).at[idx].set(True)
    pad_ch = jnp.nonzero(
        ~covered, size=TOTAL_C - c_in, fill_value=0)[0].astype(jnp.int32)
    pad_rows = (base + pad_ch[None, :]).reshape(NW, -1, CHUNK)

    x2 = x.reshape(b * c_in, hw)
    zrows = jnp.zeros((CHUNK, hw), jnp.float32)
    out2 = _sc_scatter(
        x2, dst_rows, pad_rows, zrows, b * c_in, b * (TOTAL_C - c_in), hw)
    return out2.reshape(b, TOTAL_C, h, w)


# speedup vs baseline: 1.0173x; 1.0173x over previous
import functools

import jax
import jax.numpy as jnp
from jax import lax
from jax.experimental import pallas as pl
from jax.experimental.pallas import tpu as pltpu
from jax.experimental.pallas import tpu_sc as plsc

TOTAL_C = 256

NC = 2
NS = 16
NW = NC * NS

CHUNK = 16
ZCHUNK = 8


def _sc_scatter(x2, dst_idx, pad_idx, zrows, n_rows, n_pad_rows, hw):
    rows_per_w = n_rows // NW
    prows_per_w = n_pad_rows // NW
    n_chunks = rows_per_w // CHUNK
    n_pchunks = prows_per_w // ZCHUNK

    mesh = plsc.VectorSubcoreMesh(core_axis_name="c", subcore_axis_name="s")

    @functools.partial(
        pl.kernel,
        mesh=mesh,
        compiler_params=pltpu.CompilerParams(use_tc_tiling_on_sc=False),
        out_type=jax.ShapeDtypeStruct((n_rows + n_pad_rows, hw), jnp.float32),
        scratch_types=[
            pltpu.VMEM((n_chunks, CHUNK), jnp.int32),
            pltpu.VMEM((n_pchunks, ZCHUNK), jnp.int32),
            pltpu.VMEM((CHUNK, hw), jnp.float32),
            pltpu.VMEM((CHUNK, hw), jnp.float32),
            pltpu.VMEM((ZCHUNK, hw), jnp.float32),
            pltpu.SemaphoreType.DMA,
            pltpu.SemaphoreType.DMA,
            pltpu.SemaphoreType.DMA,
            pltpu.SemaphoreType.DMA,
            pltpu.SemaphoreType.DMA,
        ],
    )
    def k(x_hbm, dsti_hbm, padi_hbm, z_hbm, out_hbm,
          idx_v, pidx_v, buf0, buf1, zbuf, gs0, gs1, ss0, ss1, zsem):
        wid = lax.axis_index("s") * NC + lax.axis_index("c")
        buf = (buf0, buf1)
        gsem = (gs0, gs1)
        ssem = (ss0, ss1)
        row0 = wid * rows_per_w

        pltpu.sync_copy(dsti_hbm.at[wid], idx_v)
        pltpu.sync_copy(padi_hbm.at[wid], pidx_v)
        pltpu.sync_copy(z_hbm, zbuf)

        zh = [
            pltpu.async_copy(zbuf, out_hbm.at[pidx_v.at[j]], zsem)
            for j in range(n_pchunks)
        ]

        gh = {}
        sh = {}
        gh[0] = pltpu.async_copy(
            x_hbm.at[pl.ds(row0, CHUNK)], buf[0], gsem[0])
        for j in range(n_chunks):
            cur = j & 1
            gh[j].wait()
            sh[j] = pltpu.async_copy(
                buf[cur], out_hbm.at[idx_v.at[j]], ssem[cur])
            if j + 1 < n_chunks:
                if j >= 1:
                    sh[j - 1].wait()
                gh[j + 1] = pltpu.async_copy(
                    x_hbm.at[pl.ds(row0 + (j + 1) * CHUNK, CHUNK)],
                    buf[1 - cur], gsem[1 - cur])
        if n_chunks >= 2:
            sh[n_chunks - 2].wait()
        sh[n_chunks - 1].wait()
        for h in zh:
            h.wait()

    return k(x2, dst_idx, pad_idx, zrows)


def kernel(x, conv_forward_indices):
    b, c_in, h, w = x.shape
    hw = h * w
    idx = conv_forward_indices.astype(jnp.int32)

    base = jnp.arange(b, dtype=jnp.int32)[:, None] * TOTAL_C
    dst_rows = (base + idx[None, :]).reshape(NW, -1, CHUNK)

    covered = jnp.zeros((TOTAL_C,), jnp.bool_).at[idx].set(True)
    pad_ch = jnp.nonzero(
        ~covered, size=TOTAL_C - c_in, fill_value=0)[0].astype(jnp.int32)
    pad_rows = (base + pad_ch[None, :]).reshape(NW, -1, ZCHUNK)

    x2 = x.reshape(b * c_in, hw)
    zrows = jnp.zeros((ZCHUNK, hw), jnp.float32)
    out2 = _sc_scatter(
        x2, dst_rows, pad_rows, zrows, b * c_in, b * (TOTAL_C - c_in), hw)
    return out2.reshape(b, TOTAL_C, h, w)
